# baseline (speedup 1.0000x reference)
import jax
import jax.numpy as jnp
from jax.experimental import pallas as pl
from jax.experimental.pallas import tpu as pltpu

B, H, L, D, C = 2, 16, 4096, 64, 512
COMMITMENT = 0.0001
HB = 2


def _fused_kernel(x_ref, m_ref, dists_ref, part_ref):
    acc = jnp.zeros((1, 1), jnp.float32)
    for t in range(HB):
        xb = x_ref[0, t]
        m = m_ref[t]
        n2 = jnp.sum(xb * xb, axis=-1, keepdims=True)
        xn = xb / jnp.maximum(jnp.sqrt(n2), 1e-12)
        d = jax.lax.dot_general(
            xn, m, (((1,), (1,)), ((), ())),
            preferred_element_type=jnp.float32)
        dists_ref[0, t] = d
        maxv = jnp.max(d, axis=-1, keepdims=True)
        m2 = jnp.sum(m * m, axis=-1)
        m2row = jnp.max(
            jnp.where(d == maxv, m2[None, :], -jnp.inf), axis=-1)
        xn2 = jnp.sum(xn * xn, axis=-1)
        acc = acc + (jnp.sum(xn2) - 2.0 * jnp.sum(maxv)
                     + jnp.sum(m2row)).reshape(1, 1)

    @pl.when(pl.program_id(0) == 0)
    def _init():
        part_ref[0] = jnp.zeros((1, 1), jnp.float32)

    part_ref[0] += acc


@jax.jit
def kernel(x, means):
    G = B * H // HB
    HG = H // HB
    dists, partials = pl.pallas_call(
        _fused_kernel,
        grid=(G,),
        in_specs=[
            pl.BlockSpec((1, HB, L, D), lambda i: (i // HG, i % HG, 0, 0)),
            pl.BlockSpec((HB, C, D), lambda i: (i % HG, 0, 0)),
        ],
        out_specs=[
            pl.BlockSpec((1, HB, L, C), lambda i: (i // HG, i % HG, 0, 0)),
            pl.BlockSpec((1, 1, 1), lambda i: (0, 0, 0)),
        ],
        out_shape=[
            jax.ShapeDtypeStruct((B, H, L, C), jnp.float32),
            jax.ShapeDtypeStruct((1, 1, 1), jnp.float32),
        ],
        compiler_params=pltpu.CompilerParams(
            dimension_semantics=("arbitrary",)),
    )(x, means)
    loss = jnp.sum(partials) * (COMMITMENT / (B * H * L * D))
    return dists, loss
